# Initial kernel scaffold; baseline (speedup 1.0000x reference)
import jax
import jax.numpy as jnp
from jax.experimental import pallas as pl


def kernel(x, W, b):
    raise NotImplementedError("write your pallas kernel here")


# speedup vs baseline: 3.0556x; 3.0556x over previous
import functools

import jax
import jax.numpy as jnp
from jax.experimental import pallas as pl
from jax.experimental.pallas import tpu as pltpu

_SEARCH_ITERS = 30


def _tc_kernel(x_ref, w_ref, b_ref, out_ref, scores_ref, *, k):
    i = pl.program_id(0)
    nb = pl.num_programs(0)
    xb = x_ref[0]
    s = jnp.dot(w_ref[...], xb, preferred_element_type=jnp.float32)
    scores_ref[pl.ds(i, 1), :] = jnp.abs(s + b_ref[0, 0])

    @pl.when(i == nb - 1)
    def _finalize():
        sa = scores_ref[...]
        hi0 = jnp.max(sa, axis=1, keepdims=True)
        lo0 = jnp.zeros_like(hi0)

        def body(_, carry):
            lo, hi = carry
            mid = (lo + hi) * 0.5
            cnt = jnp.sum((sa >= mid).astype(jnp.float32), axis=1,
                          keepdims=True)
            take = cnt >= float(k)
            return jnp.where(take, mid, lo), jnp.where(take, hi, mid)

        lo, hi = jax.lax.fori_loop(0, _SEARCH_ITERS, body, (lo0, hi0))
        t = lo
        gt = sa > t
        cnt_gt = jnp.sum(gt.astype(jnp.float32), axis=1, keepdims=True)
        sum_gt = jnp.sum(jnp.where(gt, sa, 0.0), axis=1, keepdims=True)
        total = sum_gt + (float(k) - cnt_gt) * t
        out_ref[...] = total * (1.0 / float(k))


@functools.partial(jax.jit, static_argnames=("interpret",))
def kernel(x, W, b, interpret=False):
    B, C, N = x.shape
    k = max(int(N * 0.1), 1)
    w2 = W.reshape(1, C)
    b2 = b.reshape(1, 1)
    return pl.pallas_call(
        functools.partial(_tc_kernel, k=k),
        grid=(B,),
        in_specs=[
            pl.BlockSpec((1, C, N), lambda i: (i, 0, 0)),
            pl.BlockSpec((1, C), lambda i: (0, 0)),
            pl.BlockSpec(memory_space=pltpu.SMEM),
        ],
        out_specs=pl.BlockSpec((B, 1), lambda i: (0, 0)),
        out_shape=jax.ShapeDtypeStruct((B, 1), jnp.float32),
        scratch_shapes=[pltpu.VMEM((B, N), jnp.float32)],
        interpret=interpret,
    )(x, w2, b2)
